# baseline (speedup 1.0000x reference)
import functools

import jax
import jax.numpy as jnp
from jax import lax
from jax.experimental import pallas as pl
from jax.experimental.pallas import tpu as pltpu
from jax.experimental.pallas import tpu_sc as plsc

BATCH = 4096
SEQ = 200
EMBED_DIM = 64
HIDDEN = 256
NUM_CLASSES = 128
EPS = 1e-5

NC = 2
NS = 16
NW = NC * NS
SAMPLES_PER_W = BATCH // NW
IDX_PER_W = SAMPLES_PER_W * SEQ
CHUNK = 128
NCHUNK = IDX_PER_W // CHUNK


def _sc_pool(table, flat_idx, dst):
    mesh = plsc.VectorSubcoreMesh(core_axis_name="c", subcore_axis_name="s")

    @functools.partial(
        pl.kernel,
        out_type=jax.ShapeDtypeStruct((BATCH, EMBED_DIM), jnp.float32),
        mesh=mesh,
        compiler_params=pltpu.CompilerParams(use_tc_tiling_on_sc=False),
        scratch_types=[
            pltpu.VMEM((CHUNK,), jnp.int32),
            pltpu.VMEM((NCHUNK, CHUNK), jnp.int32),
            pltpu.VMEM((CHUNK, EMBED_DIM), jnp.float32),
            pltpu.VMEM_SHARED((BATCH // NC, EMBED_DIM), jnp.float32),
            pltpu.SemaphoreType.DMA,
        ],
    )
    def k(table_hbm, idx_hbm, dst_hbm, out_hbm, idx_v, dst_v, rows_v, acc, sem):
        cid = lax.axis_index("c")
        sid = lax.axis_index("s")
        sample_base = (cid * NS + sid) * SAMPLES_PER_W
        base = sample_base * SEQ
        slab = pl.ds(sid * SAMPLES_PER_W, SAMPLES_PER_W)

        zeros = jnp.zeros((16,), jnp.float32)

        @pl.loop(0, CHUNK)
        def _(r):
            for c0 in range(0, EMBED_DIM, 16):
                rows_v[r, pl.ds(c0, 16)] = zeros

        pltpu.sync_copy(rows_v, acc.at[slab])
        pltpu.sync_copy(dst_hbm.at[sid], dst_v)

        @pl.loop(0, NCHUNK)
        def _(c):
            pltpu.sync_copy(idx_hbm.at[pl.ds(base + c * CHUNK, CHUNK)], idx_v)
            pltpu.async_copy(table_hbm.at[idx_v], rows_v, sem).wait()
            pltpu.sync_copy(rows_v, acc.at[dst_v.at[c]], add=True)

        pltpu.sync_copy(acc.at[slab],
                        out_hbm.at[pl.ds(sample_base, SAMPLES_PER_W)])

    return k(table, flat_idx, dst)


def _tc_mlp(pooled, W1, b1, g1, be1, W2, b2, g2, be2):
    def body(p_ref, w1_ref, b1_ref, g1_ref, be1_ref,
             w2_ref, b2_ref, g2_ref, be2_ref, o_ref):
        p = p_ref[...] * (1.0 / SEQ)
        h = lax.dot_general(p, w1_ref[...], (((1,), (1,)), ((), ())),
                            preferred_element_type=jnp.float32) + b1_ref[...]
        mu = jnp.mean(h, axis=0, keepdims=True)
        var = jnp.mean((h - mu) ** 2, axis=0, keepdims=True)
        h = g1_ref[...] * (h - mu) * lax.rsqrt(var + EPS) + be1_ref[...]
        h = jnp.maximum(h, 0.0)
        o = lax.dot_general(h, w2_ref[...], (((1,), (1,)), ((), ())),
                            preferred_element_type=jnp.float32) + b2_ref[...]
        mu2 = jnp.mean(o, axis=0, keepdims=True)
        var2 = jnp.mean((o - mu2) ** 2, axis=0, keepdims=True)
        o_ref[...] = g2_ref[...] * (o - mu2) * lax.rsqrt(var2 + EPS) + be2_ref[...]

    return pl.pallas_call(
        body,
        out_shape=jax.ShapeDtypeStruct((BATCH, NUM_CLASSES), jnp.float32),
    )(pooled, W1, b1.reshape(1, -1), g1.reshape(1, -1), be1.reshape(1, -1),
      W2, b2.reshape(1, -1), g2.reshape(1, -1), be2.reshape(1, -1))


def kernel(x, table, W1, b1, g1, be1, W2, b2, g2, be2):
    flat = x.reshape(-1).astype(jnp.int32)
    local = jnp.arange(IDX_PER_W, dtype=jnp.int32) // SEQ
    dst = (jnp.arange(NS, dtype=jnp.int32)[:, None] * SAMPLES_PER_W
           + local[None, :]).reshape(NS, NCHUNK, CHUNK)
    pooled = _sc_pool(table, flat, dst)
    return _tc_mlp(pooled, W1, b1, g1, be1, W2, b2, g2, be2)


# speedup vs baseline: 1.2507x; 1.2507x over previous
import functools

import jax
import jax.numpy as jnp
from jax import lax
from jax.experimental import pallas as pl
from jax.experimental.pallas import tpu as pltpu
from jax.experimental.pallas import tpu_sc as plsc

BATCH = 4096
SEQ = 200
EMBED_DIM = 64
HIDDEN = 256
NUM_CLASSES = 128
EPS = 1e-5

NC = 2
NS = 16
NW = NC * NS
SAMPLES_PER_W = BATCH // NW
IDX_PER_W = SAMPLES_PER_W * SEQ
CHUNK = 128
NCHUNK = IDX_PER_W // CHUNK


def _sc_pool(table, flat_idx, dst):
    mesh = plsc.VectorSubcoreMesh(core_axis_name="c", subcore_axis_name="s")

    NBUF = 4

    @functools.partial(
        pl.kernel,
        out_type=jax.ShapeDtypeStruct((BATCH, EMBED_DIM), jnp.float32),
        mesh=mesh,
        compiler_params=pltpu.CompilerParams(use_tc_tiling_on_sc=False),
        scratch_types=[
            pltpu.VMEM((IDX_PER_W,), jnp.int32),
            pltpu.VMEM((NCHUNK, CHUNK), jnp.int32),
            pltpu.VMEM((NBUF, CHUNK, EMBED_DIM), jnp.float32),
            pltpu.VMEM_SHARED((BATCH // NC, EMBED_DIM), jnp.float32),
            pltpu.SemaphoreType.DMA,
            pltpu.SemaphoreType.DMA,
            pltpu.SemaphoreType.DMA,
            pltpu.SemaphoreType.DMA,
        ],
    )
    def k(table_hbm, idx_hbm, dst_hbm, out_hbm,
          idx_all, dst_v, rows, acc, s0, s1, s2, s3):
        gsem = [s0, s1, s2, s3]
        cid = lax.axis_index("c")
        sid = lax.axis_index("s")
        sample_base = (cid * NS + sid) * SAMPLES_PER_W
        base = sample_base * SEQ
        slab = pl.ds(sid * SAMPLES_PER_W, SAMPLES_PER_W)

        zeros = jnp.zeros((16,), jnp.float32)

        @pl.loop(0, CHUNK)
        def _(r):
            for c0 in range(0, EMBED_DIM, 16):
                rows[0, r, pl.ds(c0, 16)] = zeros

        pltpu.sync_copy(rows.at[0], acc.at[slab])
        pltpu.sync_copy(idx_hbm.at[pl.ds(base, IDX_PER_W)], idx_all)
        pltpu.sync_copy(dst_hbm.at[sid], dst_v)

        def fire(c, b):
            pltpu.async_copy(
                table_hbm.at[idx_all.at[pl.ds(c * CHUNK, CHUNK)]],
                rows.at[b], gsem[b])

        def drain(c, b):
            pltpu.make_async_copy(
                table_hbm.at[idx_all.at[pl.ds(0, CHUNK)]],
                rows.at[b], gsem[b]).wait()
            pltpu.sync_copy(rows.at[b], acc.at[dst_v.at[c]], add=True)

        for b in range(NBUF):
            fire(b, b)

        @pl.loop(0, NCHUNK - NBUF, step=NBUF)
        def _(c0):
            for b in range(NBUF):
                drain(c0 + b, b)
                fire(c0 + b + NBUF, b)

        for b in range(NBUF):
            drain(NCHUNK - NBUF + b, b)

        pltpu.sync_copy(acc.at[slab],
                        out_hbm.at[pl.ds(sample_base, SAMPLES_PER_W)])

    return k(table, flat_idx, dst)


def _tc_mlp(pooled, W1, b1, g1, be1, W2, b2, g2, be2):
    def body(p_ref, w1_ref, b1_ref, g1_ref, be1_ref,
             w2_ref, b2_ref, g2_ref, be2_ref, o_ref):
        p = p_ref[...] * (1.0 / SEQ)
        h = lax.dot_general(p, w1_ref[...], (((1,), (1,)), ((), ())),
                            preferred_element_type=jnp.float32) + b1_ref[...]
        mu = jnp.mean(h, axis=0, keepdims=True)
        var = jnp.mean((h - mu) ** 2, axis=0, keepdims=True)
        h = g1_ref[...] * (h - mu) * lax.rsqrt(var + EPS) + be1_ref[...]
        h = jnp.maximum(h, 0.0)
        o = lax.dot_general(h, w2_ref[...], (((1,), (1,)), ((), ())),
                            preferred_element_type=jnp.float32) + b2_ref[...]
        mu2 = jnp.mean(o, axis=0, keepdims=True)
        var2 = jnp.mean((o - mu2) ** 2, axis=0, keepdims=True)
        o_ref[...] = g2_ref[...] * (o - mu2) * lax.rsqrt(var2 + EPS) + be2_ref[...]

    return pl.pallas_call(
        body,
        out_shape=jax.ShapeDtypeStruct((BATCH, NUM_CLASSES), jnp.float32),
    )(pooled, W1, b1.reshape(1, -1), g1.reshape(1, -1), be1.reshape(1, -1),
      W2, b2.reshape(1, -1), g2.reshape(1, -1), be2.reshape(1, -1))


def kernel(x, table, W1, b1, g1, be1, W2, b2, g2, be2):
    flat = x.reshape(-1).astype(jnp.int32)
    local = jnp.arange(IDX_PER_W, dtype=jnp.int32) // SEQ
    dst = (jnp.arange(NS, dtype=jnp.int32)[:, None] * SAMPLES_PER_W
           + local[None, :]).reshape(NS, NCHUNK, CHUNK)
    pooled = _sc_pool(table, flat, dst)
    return _tc_mlp(pooled, W1, b1, g1, be1, W2, b2, g2, be2)


# speedup vs baseline: 1.2581x; 1.0059x over previous
import functools

import jax
import jax.numpy as jnp
from jax import lax
from jax.experimental import pallas as pl
from jax.experimental.pallas import tpu as pltpu
from jax.experimental.pallas import tpu_sc as plsc

BATCH = 4096
SEQ = 200
EMBED_DIM = 64
HIDDEN = 256
NUM_CLASSES = 128
EPS = 1e-5

NC = 2
NS = 16
NW = NC * NS
SAMPLES_PER_W = BATCH // NW
IDX_PER_W = SAMPLES_PER_W * SEQ
CHUNK = 128
NCHUNK = IDX_PER_W // CHUNK


def _sc_pool(table, flat_idx, dst):
    mesh = plsc.VectorSubcoreMesh(core_axis_name="c", subcore_axis_name="s")

    NBUF = 4

    @functools.partial(
        pl.kernel,
        out_type=jax.ShapeDtypeStruct((BATCH, EMBED_DIM), jnp.float32),
        mesh=mesh,
        compiler_params=pltpu.CompilerParams(use_tc_tiling_on_sc=False),
        scratch_types=[
            pltpu.VMEM((NCHUNK, CHUNK), jnp.int32),
            pltpu.VMEM((NCHUNK + 1, CHUNK), jnp.int32),
            pltpu.VMEM((NBUF, CHUNK, EMBED_DIM), jnp.float32),
            pltpu.VMEM((CHUNK, EMBED_DIM), jnp.float32),
            pltpu.VMEM_SHARED((BATCH // NC, EMBED_DIM), jnp.float32),
            pltpu.SemaphoreType.DMA,
            pltpu.SemaphoreType.DMA,
            pltpu.SemaphoreType.DMA,
            pltpu.SemaphoreType.DMA,
        ],
    )
    def k(table_hbm, idx_hbm, dst_hbm, out_hbm,
          idx_all, dst_v, rows, zbuf, acc, s0, s1, s2, s3):
        gsem = [s0, s1, s2, s3]
        cid = lax.axis_index("c")
        sid = lax.axis_index("s")
        sample_base = (cid * NS + sid) * SAMPLES_PER_W
        slab = pl.ds(sid * SAMPLES_PER_W, SAMPLES_PER_W)

        zeros = jnp.zeros((16,), jnp.float32)

        @pl.loop(0, CHUNK)
        def _(r):
            for c0 in range(0, EMBED_DIM, 16):
                zbuf[r, pl.ds(c0, 16)] = zeros

        pltpu.sync_copy(zbuf, acc.at[slab])
        pltpu.sync_copy(idx_hbm.at[cid * NS + sid], idx_all)
        pltpu.sync_copy(dst_hbm.at[sid], dst_v)

        def fire(c, b):
            pltpu.async_copy(table_hbm.at[idx_all.at[c]], rows.at[b], gsem[b])

        def drain(c, b):
            pltpu.make_async_copy(
                table_hbm.at[idx_all.at[0]],
                rows.at[b], gsem[b]).wait()
            pltpu.sync_copy(rows.at[b], acc.at[dst_v.at[c]], add=True)

        for b in range(NBUF):
            fire(b, b)

        @pl.loop(0, NCHUNK - NBUF, step=NBUF)
        def _(c0):
            for b in range(NBUF):
                drain(c0 + b, b)
                fire(c0 + b + NBUF, b)

        for b in range(NBUF):
            drain(NCHUNK - NBUF + b, b)

        pltpu.sync_copy(zbuf, acc.at[dst_v.at[NCHUNK]], add=True)

        pltpu.sync_copy(acc.at[slab],
                        out_hbm.at[pl.ds(sample_base, SAMPLES_PER_W)])

    return k(table, flat_idx, dst)


def _tc_mlp(pooled, W1, b1, g1, be1, W2, b2, g2, be2):
    def body(p_ref, w1_ref, b1_ref, g1_ref, be1_ref,
             w2_ref, b2_ref, g2_ref, be2_ref, o_ref):
        p = p_ref[...] * (1.0 / SEQ)
        h = lax.dot_general(p, w1_ref[...], (((1,), (1,)), ((), ())),
                            preferred_element_type=jnp.float32) + b1_ref[...]
        mu = jnp.mean(h, axis=0, keepdims=True)
        var = jnp.mean((h - mu) ** 2, axis=0, keepdims=True)
        h = g1_ref[...] * (h - mu) * lax.rsqrt(var + EPS) + be1_ref[...]
        h = jnp.maximum(h, 0.0)
        o = lax.dot_general(h, w2_ref[...], (((1,), (1,)), ((), ())),
                            preferred_element_type=jnp.float32) + b2_ref[...]
        mu2 = jnp.mean(o, axis=0, keepdims=True)
        var2 = jnp.mean((o - mu2) ** 2, axis=0, keepdims=True)
        o_ref[...] = g2_ref[...] * (o - mu2) * lax.rsqrt(var2 + EPS) + be2_ref[...]

    return pl.pallas_call(
        body,
        out_shape=jax.ShapeDtypeStruct((BATCH, NUM_CLASSES), jnp.float32),
    )(pooled, W1, b1.reshape(1, -1), g1.reshape(1, -1), be1.reshape(1, -1),
      W2, b2.reshape(1, -1), g2.reshape(1, -1), be2.reshape(1, -1))


def kernel(x, table, W1, b1, g1, be1, W2, b2, g2, be2):
    flat = x.reshape(NW, NCHUNK, CHUNK).astype(jnp.int32)
    local = jnp.arange(IDX_PER_W, dtype=jnp.int32) // SEQ
    dst = (jnp.arange(NS, dtype=jnp.int32)[:, None] * SAMPLES_PER_W
           + local[None, :]).reshape(NS, NCHUNK, CHUNK)
    flush = (jnp.arange(NS, dtype=jnp.int32)[:, None] * SAMPLES_PER_W
             + jnp.arange(CHUNK, dtype=jnp.int32)[None, :])
    dst = jnp.concatenate([dst, flush[:, None, :]], axis=1)
    pooled = _sc_pool(table, flat, dst)
    return _tc_mlp(pooled, W1, b1, g1, be1, W2, b2, g2, be2)


# speedup vs baseline: 1.3594x; 1.0805x over previous
import functools

import jax
import jax.numpy as jnp
from jax import lax
from jax.experimental import pallas as pl
from jax.experimental.pallas import tpu as pltpu
from jax.experimental.pallas import tpu_sc as plsc

BATCH = 4096
SEQ = 200
EMBED_DIM = 64
HIDDEN = 256
NUM_CLASSES = 128
EPS = 1e-5

NC = 2
NS = 16
NW = NC * NS
SAMPLES_PER_W = BATCH // NW
IDX_PER_W = SAMPLES_PER_W * SEQ
CHUNK = 128
NCHUNK = IDX_PER_W // CHUNK


def _sc_pool(table, flat_idx, dst):
    mesh = plsc.VectorSubcoreMesh(core_axis_name="c", subcore_axis_name="s")

    NBUF = 8

    @functools.partial(
        pl.kernel,
        out_type=jax.ShapeDtypeStruct((BATCH, EMBED_DIM), jnp.float32),
        mesh=mesh,
        compiler_params=pltpu.CompilerParams(use_tc_tiling_on_sc=False),
        scratch_types=[
            pltpu.VMEM((NCHUNK, CHUNK), jnp.int32),
            pltpu.VMEM((CHUNK,), jnp.int32),
            pltpu.VMEM((NBUF, CHUNK, EMBED_DIM), jnp.float32),
            pltpu.VMEM((CHUNK, EMBED_DIM), jnp.float32),
            pltpu.VMEM_SHARED((BATCH // NC, EMBED_DIM), jnp.float32),
        ] + [pltpu.SemaphoreType.DMA] * NBUF,
    )
    def k(table_hbm, idx_hbm, dst_hbm, out_hbm,
          idx_all, dst_v, rows, zbuf, acc, *gsem):
        cid = lax.axis_index("c")
        sid = lax.axis_index("s")
        sample_base = (cid * NS + sid) * SAMPLES_PER_W
        slab = pl.ds(sid * SAMPLES_PER_W, SAMPLES_PER_W)

        zeros = jnp.zeros((16,), jnp.float32)

        @pl.loop(0, CHUNK)
        def _(r):
            for c0 in range(0, EMBED_DIM, 16):
                zbuf[r, pl.ds(c0, 16)] = zeros

        pltpu.sync_copy(zbuf, acc.at[slab])
        pltpu.sync_copy(idx_hbm.at[cid * NS + sid], idx_all)
        pltpu.sync_copy(dst_hbm.at[sid], dst_v)

        def fire(c, b):
            pltpu.async_copy(table_hbm.at[idx_all.at[c]], rows.at[b], gsem[b])

        def drain(c, b):
            del c
            pltpu.make_async_copy(
                table_hbm.at[idx_all.at[0]],
                rows.at[b], gsem[b]).wait()
            pltpu.sync_copy(rows.at[b], acc.at[dst_v], add=True)

        for b in range(NBUF):
            fire(b, b)

        @pl.loop(0, NCHUNK - NBUF, step=NBUF)
        def _(c0):
            for b in range(NBUF):
                drain(c0 + b, b)
                fire(c0 + b + NBUF, b)

        for b in range(NBUF):
            drain(NCHUNK - NBUF + b, b)

        pltpu.sync_copy(zbuf, acc.at[dst_v], add=True)

        pltpu.sync_copy(acc.at[slab],
                        out_hbm.at[pl.ds(sample_base, SAMPLES_PER_W)])

    return k(table, flat_idx, dst)


def _tc_mlp(pooled, W1, b1, g1, be1, W2, b2, g2, be2):
    def body(p_ref, w1_ref, b1_ref, g1_ref, be1_ref,
             w2_ref, b2_ref, g2_ref, be2_ref, o_ref):
        p = p_ref[...] * (1.0 / SEQ)
        h = lax.dot_general(p, w1_ref[...], (((1,), (1,)), ((), ())),
                            preferred_element_type=jnp.float32) + b1_ref[...]
        mu = jnp.mean(h, axis=0, keepdims=True)
        var = jnp.mean((h - mu) ** 2, axis=0, keepdims=True)
        h = g1_ref[...] * (h - mu) * lax.rsqrt(var + EPS) + be1_ref[...]
        h = jnp.maximum(h, 0.0)
        o = lax.dot_general(h, w2_ref[...], (((1,), (1,)), ((), ())),
                            preferred_element_type=jnp.float32) + b2_ref[...]
        mu2 = jnp.mean(o, axis=0, keepdims=True)
        var2 = jnp.mean((o - mu2) ** 2, axis=0, keepdims=True)
        o_ref[...] = g2_ref[...] * (o - mu2) * lax.rsqrt(var2 + EPS) + be2_ref[...]

    return pl.pallas_call(
        body,
        out_shape=jax.ShapeDtypeStruct((BATCH, NUM_CLASSES), jnp.float32),
    )(pooled, W1, b1.reshape(1, -1), g1.reshape(1, -1), be1.reshape(1, -1),
      W2, b2.reshape(1, -1), g2.reshape(1, -1), be2.reshape(1, -1))


def kernel(x, table, W1, b1, g1, be1, W2, b2, g2, be2):
    flat = (x.reshape(NW, SAMPLES_PER_W, SEQ)
            .transpose(0, 2, 1)
            .astype(jnp.int32))
    dst = (jnp.arange(NS, dtype=jnp.int32)[:, None] * SAMPLES_PER_W
           + jnp.arange(CHUNK, dtype=jnp.int32)[None, :])
    pooled = _sc_pool(table, flat, dst)
    return _tc_mlp(pooled, W1, b1, g1, be1, W2, b2, g2, be2)
